# baseline (speedup 1.0000x reference)
import functools

import numpy as np
import jax
import jax.numpy as jnp
from jax import lax
from jax.experimental import pallas as pl
from jax.experimental.pallas import tpu as pltpu
from jax.experimental.pallas import tpu_sc as plsc

VOCAB = 100000
SEQ_LEN = 2048
D_MODEL = 768
N_BASE = 10000
BATCH = 4

NUM_CORES = 2
NUM_SUBCORES = 16
LANES = 16
NW = NUM_CORES * NUM_SUBCORES
TOTAL = BATCH * SEQ_LEN
S_PER_W = SEQ_LEN // NW
PER_W = S_PER_W * BATCH
SUB = 16
NGRP = S_PER_W // SUB
NVREG = D_MODEL // LANES


def _positional_encoding():
    depth = D_MODEL // 2
    positions = np.arange(SEQ_LEN)[:, np.newaxis]
    depths = np.arange(depth)[np.newaxis, :] / depth
    angle_rads = positions * (1.0 / N_BASE ** depths)
    enc = np.zeros((SEQ_LEN, D_MODEL), dtype=np.float32)
    enc[:, 0::2] = np.sin(angle_rads)
    enc[:, 1::2] = np.cos(angle_rads)
    return enc


_POS_ENC = _positional_encoding()

_mesh = plsc.VectorSubcoreMesh(core_axis_name="c", subcore_axis_name="s")


@functools.partial(
    pl.kernel,
    out_type=jax.ShapeDtypeStruct((TOTAL, D_MODEL), jnp.float32),
    mesh=_mesh,
    scratch_types=[
        pltpu.VMEM((BATCH, S_PER_W), jnp.int32),
        [pltpu.VMEM((SUB, D_MODEL), jnp.float32) for _ in range(2)],
        [[pltpu.VMEM((SUB, D_MODEL), jnp.float32) for _ in range(2)]
         for _ in range(BATCH)],
        pltpu.SemaphoreType.DMA((2,)),
        pltpu.SemaphoreType.DMA((BATCH, 2)),
        pltpu.SemaphoreType.DMA((BATCH, 2)),
        pltpu.SemaphoreType.DMA,
    ],
)
def _emb_lookup(idx_hbm, table_hbm, pos_hbm, out_hbm,
                idx_v, pos_v, rows_v, sem_p, sem_g, sem_s, sem_i):
    wid = lax.axis_index("s") * NUM_CORES + lax.axis_index("c")
    s0 = wid * S_PER_W

    idx_cps = [
        pltpu.async_copy(idx_hbm.at[b, pl.ds(s0, S_PER_W)], idx_v.at[b], sem_i)
        for b in range(BATCH)]

    def gather(g, b):
        src = table_hbm.at[idx_v.at[b, pl.ds(g * SUB, SUB)]]
        return pltpu.async_copy(src, rows_v[b][g % 2], sem_g.at[b, g % 2])

    def pos_load(g):
        return pltpu.async_copy(pos_hbm.at[pl.ds(s0 + g * SUB, SUB), :],
                                pos_v[g % 2], sem_p.at[g % 2])

    pos_pending = {g: pos_load(g) for g in range(2)}
    for cp in idx_cps:
        cp.wait()
    pending = {g: [gather(g, b) for b in range(BATCH)] for g in range(2)}
    stores = {}
    for g in range(NGRP):
        q = g % 2
        if g >= 1 and g + 1 < NGRP:
            pos_pending[g + 1] = pos_load(g + 1)
            for d in stores.pop(g - 1):
                d.wait()
            pending[g + 1] = [gather(g + 1, b) for b in range(BATCH)]
        pos_pending.pop(g).wait()
        for d in pending.pop(g):
            d.wait()

        def _row_add(r, carry, q=q):
            for j in range(NVREG):
                sl = pl.ds(j * LANES, LANES)
                v = pos_v[q][r, sl]
                for b in range(BATCH):
                    rows_v[b][q][r, sl] = rows_v[b][q][r, sl] + v
            return carry

        lax.fori_loop(0, SUB, _row_add, 0)
        stores[g] = [
            pltpu.async_copy(
                rows_v[b][q],
                out_hbm.at[pl.ds(b * SEQ_LEN + s0 + g * SUB, SUB), :],
                sem_s.at[b, q])
            for b in range(BATCH)]

    for ds in stores.values():
        for d in ds:
            d.wait()


def kernel(x, table):
    xi = x.reshape(BATCH, SEQ_LEN).astype(jnp.int32)
    pos = jnp.asarray(_POS_ENC)
    out = _emb_lookup(xi, table, pos)
    return out.reshape(BATCH, SEQ_LEN, D_MODEL)


# speedup vs baseline: 1.0272x; 1.0272x over previous
import functools

import numpy as np
import jax
import jax.numpy as jnp
from jax import lax
from jax.experimental import pallas as pl
from jax.experimental.pallas import tpu as pltpu
from jax.experimental.pallas import tpu_sc as plsc

VOCAB = 100000
SEQ_LEN = 2048
D_MODEL = 768
N_BASE = 10000
BATCH = 4

NUM_CORES = 2
NUM_SUBCORES = 16
LANES = 16
NW = NUM_CORES * NUM_SUBCORES
TOTAL = BATCH * SEQ_LEN
S_PER_W = SEQ_LEN // NW
PER_W = S_PER_W * BATCH
SUB = 8
NGRP = S_PER_W // SUB
GDEPTH = 3
NVREG = D_MODEL // LANES


def _positional_encoding():
    depth = D_MODEL // 2
    positions = np.arange(SEQ_LEN)[:, np.newaxis]
    depths = np.arange(depth)[np.newaxis, :] / depth
    angle_rads = positions * (1.0 / N_BASE ** depths)
    enc = np.zeros((SEQ_LEN, D_MODEL), dtype=np.float32)
    enc[:, 0::2] = np.sin(angle_rads)
    enc[:, 1::2] = np.cos(angle_rads)
    return enc


_POS_ENC = _positional_encoding()

_mesh = plsc.VectorSubcoreMesh(core_axis_name="c", subcore_axis_name="s")


@functools.partial(
    pl.kernel,
    out_type=jax.ShapeDtypeStruct((TOTAL, D_MODEL), jnp.float32),
    mesh=_mesh,
    scratch_types=[
        pltpu.VMEM((BATCH, S_PER_W), jnp.int32),
        [pltpu.VMEM((SUB, D_MODEL), jnp.float32) for _ in range(GDEPTH)],
        [[pltpu.VMEM((SUB, D_MODEL), jnp.float32) for _ in range(GDEPTH)]
         for _ in range(BATCH)],
        pltpu.SemaphoreType.DMA((GDEPTH,)),
        pltpu.SemaphoreType.DMA((BATCH, GDEPTH)),
        pltpu.SemaphoreType.DMA((BATCH, GDEPTH)),
        pltpu.SemaphoreType.DMA,
    ],
)
def _emb_lookup(idx_hbm, table_hbm, pos_hbm, out_hbm,
                idx_v, pos_v, rows_v, sem_p, sem_g, sem_s, sem_i):
    wid = lax.axis_index("s") * NUM_CORES + lax.axis_index("c")
    s0 = wid * S_PER_W

    idx_cps = [
        pltpu.async_copy(idx_hbm.at[b, pl.ds(s0, S_PER_W)], idx_v.at[b], sem_i)
        for b in range(BATCH)]

    def gather(g, b):
        src = table_hbm.at[idx_v.at[b, pl.ds(g * SUB, SUB)]]
        return pltpu.async_copy(src, rows_v[b][g % GDEPTH],
                                sem_g.at[b, g % GDEPTH])

    def pos_load(g):
        return pltpu.async_copy(pos_hbm.at[pl.ds(s0 + g * SUB, SUB), :],
                                pos_v[g % GDEPTH], sem_p.at[g % GDEPTH])

    pos_pending = {g: pos_load(g) for g in range(GDEPTH)}
    for cp in idx_cps:
        cp.wait()
    pending = {g: [gather(g, b) for b in range(BATCH)] for g in range(GDEPTH)}
    stores = {}
    for g in range(NGRP):
        q = g % GDEPTH
        nxt = g + GDEPTH - 1
        if g >= 1 and nxt < NGRP:
            pos_pending[nxt] = pos_load(nxt)
            for d in stores.pop(g - 1):
                d.wait()
            pending[nxt] = [gather(nxt, b) for b in range(BATCH)]
        pos_pending.pop(g).wait()
        for d in pending.pop(g):
            d.wait()

        def _row_add(r, carry, q=q):
            for j in range(NVREG):
                sl = pl.ds(j * LANES, LANES)
                v = pos_v[q][r, sl]
                for b in range(BATCH):
                    plsc.addupdate(rows_v[b][q].at[r, sl], v)
            return carry

        lax.fori_loop(0, SUB, _row_add, 0)
        stores[g] = [
            pltpu.async_copy(
                rows_v[b][q],
                out_hbm.at[pl.ds(b * SEQ_LEN + s0 + g * SUB, SUB), :],
                sem_s.at[b, q])
            for b in range(BATCH)]

    for ds in stores.values():
        for d in ds:
            d.wait()


def kernel(x, table):
    xi = x.reshape(BATCH, SEQ_LEN).astype(jnp.int32)
    pos = jnp.asarray(_POS_ENC)
    out = _emb_lookup(xi, table, pos)
    return out.reshape(BATCH, SEQ_LEN, D_MODEL)


# speedup vs baseline: 1.0338x; 1.0065x over previous
import functools

import numpy as np
import jax
import jax.numpy as jnp
from jax import lax
from jax.experimental import pallas as pl
from jax.experimental.pallas import tpu as pltpu
from jax.experimental.pallas import tpu_sc as plsc

VOCAB = 100000
SEQ_LEN = 2048
D_MODEL = 768
N_BASE = 10000
BATCH = 4

NUM_CORES = 2
NUM_SUBCORES = 16
LANES = 16
NW = NUM_CORES * NUM_SUBCORES
TOTAL = BATCH * SEQ_LEN
S_PER_W = SEQ_LEN // NW
PER_W = S_PER_W * BATCH
SUB = 16
NGRP = S_PER_W // SUB
GDEPTH = 2
NVREG = D_MODEL // LANES


def _positional_encoding():
    depth = D_MODEL // 2
    positions = np.arange(SEQ_LEN)[:, np.newaxis]
    depths = np.arange(depth)[np.newaxis, :] / depth
    angle_rads = positions * (1.0 / N_BASE ** depths)
    enc = np.zeros((SEQ_LEN, D_MODEL), dtype=np.float32)
    enc[:, 0::2] = np.sin(angle_rads)
    enc[:, 1::2] = np.cos(angle_rads)
    return enc


_POS_ENC = _positional_encoding()

_mesh = plsc.VectorSubcoreMesh(core_axis_name="c", subcore_axis_name="s")


@functools.partial(
    pl.kernel,
    out_type=jax.ShapeDtypeStruct((TOTAL, D_MODEL), jnp.float32),
    mesh=_mesh,
    scratch_types=[
        pltpu.VMEM((BATCH, S_PER_W), jnp.int32),
        [pltpu.VMEM((SUB, D_MODEL), jnp.float32) for _ in range(GDEPTH)],
        [[pltpu.VMEM((SUB, D_MODEL), jnp.float32) for _ in range(GDEPTH)]
         for _ in range(BATCH)],
        pltpu.SemaphoreType.DMA((GDEPTH,)),
        pltpu.SemaphoreType.DMA((BATCH, GDEPTH)),
        pltpu.SemaphoreType.DMA((BATCH, GDEPTH)),
        pltpu.SemaphoreType.DMA,
    ],
)
def _emb_lookup(idx_hbm, table_hbm, pos_hbm, out_hbm,
                idx_v, pos_v, rows_v, sem_p, sem_g, sem_s, sem_i):
    wid = lax.axis_index("s") * NUM_CORES + lax.axis_index("c")
    s0 = wid * S_PER_W

    idx_cps = [
        pltpu.async_copy(idx_hbm.at[b, pl.ds(s0, S_PER_W)], idx_v.at[b], sem_i)
        for b in range(BATCH)]

    def gather(g, b):
        src = table_hbm.at[idx_v.at[b, pl.ds(g * SUB, SUB)]]
        return pltpu.async_copy(src, rows_v[b][g % GDEPTH],
                                sem_g.at[b, g % GDEPTH])

    def pos_load(g):
        return pltpu.async_copy(pos_hbm.at[pl.ds(s0 + g * SUB, SUB), :],
                                pos_v[g % GDEPTH], sem_p.at[g % GDEPTH])

    pos_pending = {g: pos_load(g) for g in range(GDEPTH)}
    for cp in idx_cps:
        cp.wait()
    pending = {g: [gather(g, b) for b in range(BATCH)] for g in range(GDEPTH)}
    stores = {}
    for g in range(NGRP):
        q = g % GDEPTH
        nxt = g + GDEPTH - 1
        if g >= 1 and nxt < NGRP:
            pos_pending[nxt] = pos_load(nxt)
            for d in stores.pop(g - 1):
                d.wait()
            pending[nxt] = [gather(nxt, b) for b in range(BATCH)]
        pos_pending.pop(g).wait()
        for d in pending.pop(g):
            d.wait()

        @plsc.parallel_loop(0, SUB)
        def _row_add(r, q=q):
            for j in range(NVREG):
                sl = pl.ds(j * LANES, LANES)
                v = pos_v[q][r, sl]
                for b in range(BATCH):
                    plsc.addupdate(rows_v[b][q].at[r, sl], v)
        stores[g] = [
            pltpu.async_copy(
                rows_v[b][q],
                out_hbm.at[pl.ds(b * SEQ_LEN + s0 + g * SUB, SUB), :],
                sem_s.at[b, q])
            for b in range(BATCH)]

    for ds in stores.values():
        for d in ds:
            d.wait()


def kernel(x, table):
    xi = x.reshape(BATCH, SEQ_LEN).astype(jnp.int32)
    pos = jnp.asarray(_POS_ENC)
    out = _emb_lookup(xi, table, pos)
    return out.reshape(BATCH, SEQ_LEN, D_MODEL)


# speedup vs baseline: 1.0589x; 1.0243x over previous
import functools

import numpy as np
import jax
import jax.numpy as jnp
from jax import lax
from jax.experimental import pallas as pl
from jax.experimental.pallas import tpu as pltpu
from jax.experimental.pallas import tpu_sc as plsc

VOCAB = 100000
SEQ_LEN = 2048
D_MODEL = 768
N_BASE = 10000
BATCH = 4

NUM_CORES = 2
NUM_SUBCORES = 16
LANES = 16
NW = NUM_CORES * NUM_SUBCORES
TOTAL = BATCH * SEQ_LEN
S_PER_W = SEQ_LEN // NW
SUB = 16
NGRP = S_PER_W // SUB
GDEPTH = 2
NVREG = D_MODEL // LANES


def _positional_encoding():
    depth = D_MODEL // 2
    positions = np.arange(SEQ_LEN)[:, np.newaxis]
    depths = np.arange(depth)[np.newaxis, :] / depth
    angle_rads = positions * (1.0 / N_BASE ** depths)
    enc = np.zeros((SEQ_LEN, D_MODEL), dtype=np.float32)
    enc[:, 0::2] = np.sin(angle_rads)
    enc[:, 1::2] = np.cos(angle_rads)
    return enc


_POS_ENC = _positional_encoding()

_mesh = plsc.VectorSubcoreMesh(core_axis_name="c", subcore_axis_name="s")


@functools.partial(
    pl.kernel,
    out_type=jax.ShapeDtypeStruct((TOTAL, D_MODEL), jnp.float32),
    mesh=_mesh,
    scratch_types=[
        pltpu.VMEM((BATCH, S_PER_W), jnp.int32),
        [pltpu.VMEM((SUB, D_MODEL), jnp.float32) for _ in range(GDEPTH)],
        [[pltpu.VMEM((SUB, D_MODEL), jnp.float32) for _ in range(GDEPTH)]
         for _ in range(BATCH)],
        pltpu.SemaphoreType.DMA((GDEPTH,)),
        pltpu.SemaphoreType.DMA((BATCH, GDEPTH)),
        pltpu.SemaphoreType.DMA((BATCH, GDEPTH)),
        pltpu.SemaphoreType.DMA,
    ],
)
def _emb_lookup(idx_hbm, table_hbm, pos_hbm, out_hbm,
                idx_v, pos_v, rows_v, sem_p, sem_g, sem_s, sem_i):
    wid = lax.axis_index("s") * NUM_CORES + lax.axis_index("c")
    s0 = wid * S_PER_W

    idx_cps = [
        pltpu.async_copy(idx_hbm.at[b, pl.ds(s0, S_PER_W)], idx_v.at[b], sem_i)
        for b in range(BATCH)]

    def gather(g, b):
        src = table_hbm.at[idx_v.at[b, pl.ds(g * SUB, SUB)]]
        return pltpu.async_copy(src, rows_v[b][g % GDEPTH],
                                sem_g.at[b, g % GDEPTH])

    def pos_load(g):
        return pltpu.async_copy(pos_hbm.at[pl.ds(s0 + g * SUB, SUB), :],
                                pos_v[g % GDEPTH], sem_p.at[g % GDEPTH])

    pos_pending = {g: pos_load(g) for g in range(GDEPTH)}
    for cp in idx_cps:
        cp.wait()
    pending = {g: [gather(g, b) for b in range(BATCH)] for g in range(GDEPTH)}
    stores = {}
    for g in range(NGRP):
        q = g % GDEPTH
        nxt = g + GDEPTH - 1
        if g >= 1 and nxt < NGRP:
            pos_pending[nxt] = pos_load(nxt)
            for d in stores.pop(g - 1):
                d.wait()
            pending[nxt] = [gather(nxt, b) for b in range(BATCH)]
        pos_pending.pop(g).wait()
        for d in pending.pop(g):
            d.wait()

        def _row_add(r, carry, q=q):
            for j in range(NVREG):
                sl = pl.ds(j * LANES, LANES)
                v = pos_v[q][r, sl]
                for b in range(BATCH):
                    plsc.addupdate(rows_v[b][q].at[r, sl], v)
            return carry

        lax.fori_loop(0, SUB, _row_add, 0)
        stores[g] = [
            pltpu.async_copy(
                rows_v[b][q],
                out_hbm.at[pl.ds(b * SEQ_LEN + s0 + g * SUB, SUB), :],
                sem_s.at[b, q])
            for b in range(BATCH)]

    for ds in stores.values():
        for d in ds:
            d.wait()


def kernel(x, table):
    xi = x.reshape(BATCH, SEQ_LEN).astype(jnp.int32)
    pos = jnp.asarray(_POS_ENC)
    out = _emb_lookup(xi, table, pos)
    return out.reshape(BATCH, SEQ_LEN, D_MODEL)
